# baseline (speedup 1.0000x reference)
import functools

import jax
import jax.numpy as jnp
from jax import lax
from jax.experimental import pallas as pl
from jax.experimental.pallas import tpu as pltpu
from jax.experimental.pallas import tpu_sc as plsc

_NUM_ENTITY = 100000
_NUM_NEIGHBOR = 16
_BATCH = 4096
_NUM_CORES = 2
_NUM_SUBCORES = 16
_NUM_WORKERS = _NUM_CORES * _NUM_SUBCORES
_SEEDS_PER_W = _BATCH // _NUM_WORKERS
_HOP2_PER_W = _SEEDS_PER_W * _NUM_NEIGHBOR
_i32 = jnp.int32


_NBLK = _NUM_ENTITY // 128
_NBLK2 = 2 * _NBLK
_BLK_PER_W = (_NBLK2 + _NUM_WORKERS - 1) // _NUM_WORKERS
_LROWS = _NUM_ENTITY * _NUM_NEIGHBOR // 128
_TCOL = _NBLK * 128
_TAIL = _NUM_ENTITY - _TCOL


_SB = 512
_NSB = _NBLK // (_SB // 128)
_SB_PER_W = (_NSB + _NUM_SUBCORES - 1) // _NUM_SUBCORES


def _dt_body(t_e_hbm, t_r_hbm, o_e, o_r,
             slab_v, stage_v, bslab_v, bstage_v, tslab_v, tstage_v, sem):
    def _do_super(sb, src, dst):
        pltpu.sync_copy(src.at[:, pl.ds(sb * _SB, _SB)], slab_v)

        @plsc.parallel_loop(0, _SB, unroll=8)
        def _row(r):
            v = plsc.load_gather(
                slab_v,
                [lax.iota(_i32, 16), jnp.full((16,), r, _i32)])
            stage_v[r // 8, pl.ds((r % 8) * 16, 16)] = v

        pltpu.sync_copy(stage_v, dst.at[pl.ds(sb * (_SB // 8), _SB // 8), :])

    def _do_block(cb, src, dst):
        pltpu.sync_copy(src.at[:, pl.ds(cb * 128, 128)], bslab_v)

        @plsc.parallel_loop(0, 128, unroll=8)
        def _row(r):
            v = plsc.load_gather(
                bslab_v,
                [lax.iota(_i32, 16), jnp.full((16,), r, _i32)])
            bstage_v[r // 8, pl.ds((r % 8) * 16, 16)] = v

        pltpu.sync_copy(bstage_v, dst.at[pl.ds(cb * 16, 16), :])

    cid = lax.axis_index("c")
    sid = lax.axis_index("s")

    def _table_loop(src_hbm, dst_hbm):
        def _sbk(k, _):
            sb = k * _NUM_SUBCORES + sid

            @pl.when(sb < _NSB)
            def _():
                _do_super(sb, src_hbm, dst_hbm)

            return 0

        lax.fori_loop(0, _SB_PER_W, _sbk, 0)

        @pl.when(sid == 15)
        def _():
            _do_block(_NBLK - 1, src_hbm, dst_hbm)

    @pl.when(cid == 0)
    def _():
        _table_loop(t_e_hbm, o_e)

    @pl.when(cid == 1)
    def _():
        _table_loop(t_r_hbm, o_r)

    def _do_tail(src, dst):
        pltpu.sync_copy(src.at[:, pl.ds(_TCOL, _TAIL)], tslab_v)

        @plsc.parallel_loop(0, _TAIL, unroll=8)
        def _trow(r):
            v = plsc.load_gather(
                tslab_v,
                [lax.iota(_i32, 16), jnp.full((16,), r, _i32)])
            tstage_v[r // 8, pl.ds((r % 8) * 16, 16)] = v
        pltpu.sync_copy(tstage_v, dst.at[pl.ds(_LROWS - _TAIL // 8, _TAIL // 8), :])

    @pl.when(jnp.logical_and(cid == 0, sid == 14))
    def _():
        _do_tail(t_e_hbm, o_e)

    @pl.when(jnp.logical_and(cid == 1, sid == 14))
    def _():
        _do_tail(t_r_hbm, o_r)


_dt_call = functools.partial(
    pl.kernel,
    out_type=(
        jax.ShapeDtypeStruct((_LROWS, 128), _i32),
        jax.ShapeDtypeStruct((_LROWS, 128), _i32),
    ),
    mesh=plsc.VectorSubcoreMesh(core_axis_name="c", subcore_axis_name="s",
                                num_cores=_NUM_CORES,
                                num_subcores=_NUM_SUBCORES),
    compiler_params=pltpu.CompilerParams(use_tc_tiling_on_sc=True,
                                         needs_layout_passes=False),
    scratch_types=[
        pltpu.VMEM((16, _SB), _i32),
        pltpu.VMEM((_SB // 8, 128), _i32),
        pltpu.VMEM((16, 128), _i32),
        pltpu.VMEM((16, 128), _i32),
        pltpu.VMEM((16, _TAIL), _i32),
        pltpu.VMEM((_TAIL // 8, 128), _i32),
        pltpu.SemaphoreType.DMA,
    ],
)(_dt_body)


def _rf_body(seeds_hbm, adj_e_hbm, adj_r_hbm,
             o_e1, o_r1, o_e2, o_r2,
             seed_v, e1_v, r1_v, idx2_v, e2_v, r2_v,
             sem_e1, sem_r1, sem_e2, sem_r2):
    wid = lax.axis_index("s") * _NUM_CORES + lax.axis_index("c")
    base = wid * _SEEDS_PER_W

    pltpu.sync_copy(seeds_hbm.at[pl.ds(base, _SEEDS_PER_W)], seed_v)

    c_e1 = pltpu.async_copy(adj_e_hbm.at[seed_v], e1_v, sem_e1)
    c_r1 = pltpu.async_copy(adj_r_hbm.at[seed_v], r1_v, sem_r1)
    c_e1.wait()

    def _flat(i, _):
        idx2_v[pl.ds(i * _NUM_NEIGHBOR, _NUM_NEIGHBOR)] = e1_v[i, :]
        return 0

    lax.fori_loop(0, _SEEDS_PER_W, _flat, 0)

    c_e2 = pltpu.async_copy(adj_e_hbm.at[idx2_v], e2_v, sem_e2)
    c_r2 = pltpu.async_copy(adj_r_hbm.at[idx2_v], r2_v, sem_r2)

    pltpu.sync_copy(e1_v, o_e1.at[pl.ds(base, _SEEDS_PER_W)])
    c_r1.wait()
    pltpu.sync_copy(r1_v, o_r1.at[pl.ds(base, _SEEDS_PER_W)])

    c_e2.wait()
    pltpu.sync_copy(e2_v, o_e2.at[pl.ds(wid * _HOP2_PER_W, _HOP2_PER_W)])
    c_r2.wait()
    pltpu.sync_copy(r2_v, o_r2.at[pl.ds(wid * _HOP2_PER_W, _HOP2_PER_W)])


_N_HOP2 = _BATCH * _NUM_NEIGHBOR
_rf_call = functools.partial(
    pl.kernel,
    out_type=(
        jax.ShapeDtypeStruct((_BATCH, _NUM_NEIGHBOR), _i32),
        jax.ShapeDtypeStruct((_BATCH, _NUM_NEIGHBOR), _i32),
        jax.ShapeDtypeStruct((_N_HOP2, _NUM_NEIGHBOR), _i32),
        jax.ShapeDtypeStruct((_N_HOP2, _NUM_NEIGHBOR), _i32),
    ),
    mesh=plsc.VectorSubcoreMesh(core_axis_name="c", subcore_axis_name="s",
                                num_cores=_NUM_CORES,
                                num_subcores=_NUM_SUBCORES),
    compiler_params=pltpu.CompilerParams(use_tc_tiling_on_sc=False),
    scratch_types=[
        pltpu.VMEM((_SEEDS_PER_W,), _i32),
        pltpu.VMEM((_SEEDS_PER_W, _NUM_NEIGHBOR), _i32),
        pltpu.VMEM((_SEEDS_PER_W, _NUM_NEIGHBOR), _i32),
        pltpu.VMEM((_HOP2_PER_W,), _i32),
        pltpu.VMEM((_HOP2_PER_W, _NUM_NEIGHBOR), _i32),
        pltpu.VMEM((_HOP2_PER_W, _NUM_NEIGHBOR), _i32),
        pltpu.SemaphoreType.DMA,
        pltpu.SemaphoreType.DMA,
        pltpu.SemaphoreType.DMA,
        pltpu.SemaphoreType.DMA,
    ],
)(_rf_body)


def kernel(inputs, adj_entity, adj_relation):
    seeds = inputs.reshape(_BATCH)
    lin_e, lin_r = _dt_call(jnp.swapaxes(adj_entity, 0, 1),
                            jnp.swapaxes(adj_relation, 0, 1))
    tab_e = lin_e.reshape(_NUM_ENTITY, _NUM_NEIGHBOR)
    tab_r = lin_r.reshape(_NUM_ENTITY, _NUM_NEIGHBOR)
    e1, r1, e2, r2 = _rf_call(seeds, tab_e, tab_r)
    n2 = _NUM_NEIGHBOR * _NUM_NEIGHBOR
    return (
        (inputs, e1, e2.reshape(_BATCH, n2)),
        (r1, r2.reshape(_BATCH, n2)),
    )
